# baseline (speedup 1.0000x reference)
import jax
import jax.numpy as jnp
from jax.experimental import pallas as pl


_EPS = 1e-5
_NUM_BLOCKS = 2


def _tiny_block_kernel(x_ref, r8_ref, gs_ref, c_ref, o_ref):
    x = x_ref[...]
    h = x.shape[1]
    cat = jnp.concatenate([x, x * x], axis=1)
    t = jax.lax.dot_general(cat, r8_ref[...], (((1,), (0,)), ((), ())),
                            preferred_element_type=jnp.float32)
    mu = t[:, 0:h]
    q = t[:, h:2 * h]
    xg = t[:, 2 * h:3 * h]
    rr = jax.lax.rsqrt(q - mu * mu + _EPS)
    o_ref[...] = x + rr * (xg - mu * gs_ref[...]) + c_ref[...]


def kernel(x, routing, W, b, gamma, beta, use_boundary):
    n, h = x.shape
    bm = n // _NUM_BLOCKS

    a4 = jnp.full((h, h), 1.0 / h, dtype=jnp.float32)
    z4 = jnp.zeros((h, h), dtype=jnp.float32)
    g = gamma[:, None] * W.T
    r8 = jnp.block([[a4, z4, g], [z4, a4, z4]])
    gs = jnp.sum(g, axis=0).reshape(1, h)
    c = (beta @ W.T + b).reshape(1, h)

    return pl.pallas_call(
        _tiny_block_kernel,
        grid=(_NUM_BLOCKS,),
        in_specs=[
            pl.BlockSpec((bm, h), lambda i: (i, 0)),
            pl.BlockSpec((2 * h, 3 * h), lambda i: (0, 0)),
            pl.BlockSpec((1, h), lambda i: (0, 0)),
            pl.BlockSpec((1, h), lambda i: (0, 0)),
        ],
        out_specs=pl.BlockSpec((bm, h), lambda i: (i, 0)),
        out_shape=jax.ShapeDtypeStruct((n, h), jnp.float32),
    )(x, r8, gs, c)


# speedup vs baseline: 1.3021x; 1.3021x over previous
import jax
import jax.numpy as jnp
from jax.experimental import pallas as pl


_EPS = 1e-5
_NUM_BLOCKS = 2


def _tiny_block_kernel(x_ref, a_ref, g_ref, gs_ref, c_ref, o_ref):
    x = x_ref[...]
    a = a_ref[...]
    dims = (((1,), (0,)), ((), ()))
    mu = jax.lax.dot_general(x, a, dims, preferred_element_type=jnp.float32)
    q = jax.lax.dot_general(x * x, a, dims, preferred_element_type=jnp.float32)
    xg = jax.lax.dot_general(x, g_ref[...], dims,
                             preferred_element_type=jnp.float32)
    rr = jax.lax.rsqrt(q - mu * mu + _EPS)
    o_ref[...] = x + rr * (xg - mu * gs_ref[...]) + c_ref[...]


def kernel(x, routing, W, b, gamma, beta, use_boundary):
    n, h = x.shape
    bm = n // _NUM_BLOCKS

    a = jnp.full((h, h), 1.0 / h, dtype=jnp.float32)
    g = gamma[:, None] * W.T
    gs = jnp.sum(g, axis=0).reshape(1, h)
    c = (beta @ W.T + b).reshape(1, h)

    return pl.pallas_call(
        _tiny_block_kernel,
        grid=(_NUM_BLOCKS,),
        in_specs=[
            pl.BlockSpec((bm, h), lambda i: (i, 0)),
            pl.BlockSpec((h, h), lambda i: (0, 0)),
            pl.BlockSpec((h, h), lambda i: (0, 0)),
            pl.BlockSpec((1, h), lambda i: (0, 0)),
            pl.BlockSpec((1, h), lambda i: (0, 0)),
        ],
        out_specs=pl.BlockSpec((bm, h), lambda i: (i, 0)),
        out_shape=jax.ShapeDtypeStruct((n, h), jnp.float32),
    )(x, a, g, gs, c)


# speedup vs baseline: 1.3892x; 1.0669x over previous
import jax
import jax.numpy as jnp
from jax.experimental import pallas as pl


_EPS = 1e-5
_NUM_BLOCKS = 2


def _tiny_block_kernel(x_ref, a_ref, g_ref, c_ref, o_ref):
    x = x_ref[...]
    a = a_ref[...]
    dims = (((1,), (0,)), ((), ()))
    mu = jax.lax.dot_general(x, a, dims, preferred_element_type=jnp.float32)
    d = x - mu
    var = jax.lax.dot_general(d * d, a, dims,
                              preferred_element_type=jnp.float32)
    hn = d * jax.lax.rsqrt(var + _EPS)
    y = jax.lax.dot_general(hn, g_ref[...], dims,
                            preferred_element_type=jnp.float32)
    o_ref[...] = x + y + c_ref[...]


def kernel(x, routing, W, b, gamma, beta, use_boundary):
    n, h = x.shape
    bm = n // _NUM_BLOCKS

    a = jnp.full((h, h), 1.0 / h, dtype=jnp.float32)
    g = gamma[:, None] * W.T
    c = (beta @ W.T + b).reshape(1, h)

    return pl.pallas_call(
        _tiny_block_kernel,
        grid=(_NUM_BLOCKS,),
        in_specs=[
            pl.BlockSpec((bm, h), lambda i: (i, 0)),
            pl.BlockSpec((h, h), lambda i: (0, 0)),
            pl.BlockSpec((h, h), lambda i: (0, 0)),
            pl.BlockSpec((1, h), lambda i: (0, 0)),
        ],
        out_specs=pl.BlockSpec((bm, h), lambda i: (i, 0)),
        out_shape=jax.ShapeDtypeStruct((n, h), jnp.float32),
    )(x, a, g, c)
